# Initial kernel scaffold; baseline (speedup 1.0000x reference)
import jax
import jax.numpy as jnp
from jax.experimental import pallas as pl


def kernel(past_key_states, past_value_states, hh_scores, W1, b1, W2, b2, normalizer):
    raise NotImplementedError("write your pallas kernel here")


# speedup vs baseline: 2.3851x; 2.3851x over previous
import functools

import jax
import jax.numpy as jnp
from jax import lax
from jax.experimental import pallas as pl
from jax.experimental.pallas import tpu as pltpu
from jax.experimental.pallas import tpu_sc as plsc

B, H, S, D = 8, 12, 2048, 64
MEM_COMPRESS = 256
KEEP_HH = 256
LOCAL_LEN = 64
DIM_KV = 2 * D
HIDDEN = 512
S_NH = S - KEEP_HH
NHEAD = B * H
K1 = KEEP_HH - LOCAL_LEN
NLOC = S - LOCAL_LEN
ROWS = NHEAD * S
CHUNK = 128
NCHUNKS = ROWS // CHUNK

_HEADS_PER_BLK = 8


def _cumsum_lanes(x):
    n = x.shape[-1]
    s = 1
    while s < n:
        shifted = jnp.concatenate(
            [jnp.zeros(x.shape[:-1] + (s,), x.dtype), x[..., : n - s]], axis=-1)
        x = x + shifted
        s *= 2
    return x


def _topk_mask(bits, valid, k):
    rows = bits.shape[0]
    validc = valid.astype(jnp.int32)

    def count_ge(m):
        return jnp.sum(jnp.where(bits >= m, validc, 0), axis=-1, keepdims=True)

    def body(_, lohi):
        lo, hi = lohi
        mid = (lo + hi) // 2
        ge = count_ge(mid) >= k
        return (jnp.where(ge, mid, lo), jnp.where(ge, hi, mid))

    lo = jnp.zeros((rows, 1), jnp.int32)
    hi = jnp.full((rows, 1), 0x3F800000, jnp.int32)
    lo, hi = lax.fori_loop(0, 31, body, (lo, hi))
    t = lo
    gt = valid & (bits > t)
    eq = valid & (bits == t)
    need = k - jnp.sum(gt.astype(jnp.int32), axis=-1, keepdims=True)
    eqrank = _cumsum_lanes(eq.astype(jnp.int32))
    return gt | (eq & (eqrank <= need))


def _index_body(scores_ref, dest_ref, rw_ref):
    blk = lax.broadcasted_iota(jnp.int32, (_HEADS_PER_BLK, S), 0)
    head0 = pl.program_id(0) * _HEADS_PER_BLK
    bits = lax.bitcast_convert_type(scores_ref[...], jnp.int32)
    pos = lax.broadcasted_iota(jnp.int32, (_HEADS_PER_BLK, S), 1)

    mask1 = _topk_mask(bits, pos < NLOC, K1) | (pos >= NLOC)
    c1 = _cumsum_lanes(mask1.astype(jnp.int32))
    dest = jnp.where(mask1, c1 - 1, KEEP_HH + pos - c1)
    dest_ref[...] = dest + (head0 + blk) * S

    mask2 = _topk_mask(bits, ~mask1, MEM_COMPRESS)
    c2 = _cumsum_lanes(mask2.astype(jnp.int32))
    r = pos - c1
    mio = lax.broadcasted_iota(jnp.int32, (MEM_COMPRESS, S), 0)
    for h in range(_HEADS_PER_BLK):
        hit = mask2[h : h + 1, :] & (c2[h : h + 1, :] - 1 == mio)
        rw_ref[h, 0, :] = jnp.sum(
            jnp.where(hit, r[h : h + 1, :], 0), axis=1)


def _index_call(scores):
    grid = (NHEAD // _HEADS_PER_BLK,)
    return pl.pallas_call(
        _index_body,
        grid=grid,
        in_specs=[pl.BlockSpec((_HEADS_PER_BLK, S), lambda i: (i, 0))],
        out_specs=[
            pl.BlockSpec((_HEADS_PER_BLK, S), lambda i: (i, 0)),
            pl.BlockSpec((_HEADS_PER_BLK, 1, MEM_COMPRESS),
                         lambda i: (i, 0, 0)),
        ],
        out_shape=[
            jax.ShapeDtypeStruct((NHEAD, S), jnp.int32),
            jax.ShapeDtypeStruct((NHEAD, 1, MEM_COMPRESS), jnp.int32),
        ],
        compiler_params=pltpu.CompilerParams(
            dimension_semantics=("parallel",)),
    )(scores)


def _sc_scatter(kv, idx):
    info = plsc.get_sparse_core_info()
    nw = info.num_cores * info.num_subcores
    per_w = NCHUNKS // nw

    mesh = plsc.VectorSubcoreMesh(core_axis_name="c", subcore_axis_name="s")

    @functools.partial(
        pl.kernel,
        mesh=mesh,
        out_type=jax.ShapeDtypeStruct((ROWS, DIM_KV), jnp.float32),
        scratch_types=[
            pltpu.VMEM((CHUNK,), jnp.int32),
            pltpu.VMEM((CHUNK, DIM_KV), jnp.float32),
            pltpu.SemaphoreType.DMA,
        ],
    )
    def scatter(kv_hbm, idx_hbm, out_hbm, idx_v, buf, sem):
        wid = lax.axis_index("s") * info.num_cores + lax.axis_index("c")

        def body(j, carry):
            c = wid * per_w + j
            pltpu.sync_copy(idx_hbm.at[c], idx_v)
            pltpu.sync_copy(kv_hbm.at[pl.ds(c * CHUNK, CHUNK)], buf)
            pltpu.async_copy(buf, out_hbm.at[idx_v], sem).wait()
            return carry

        lax.fori_loop(0, per_w, body, 0)

    return scatter(kv, idx)


def _shift_rows(x, s):
    n = x.shape[0]
    if s == 0:
        return x
    if s > 0:
        return jnp.concatenate(
            [x[s:, :], jnp.zeros((s, x.shape[1]), x.dtype)], axis=0)
    return jnp.concatenate(
        [jnp.zeros((-s, x.shape[1]), x.dtype), x[: n + s, :]], axis=0)


def _conv_body(pkv_ref, rw_ref, w1_ref, w2_ref, b1_ref, b2_ref,
               nrm_ref, nk_ref, nv_ref):
    prec = lax.Precision.HIGHEST
    x = pkv_ref[0][KEEP_HH:, :]

    y1 = jnp.zeros((S_NH, HIDDEN), jnp.float32)
    for dk in range(3):
        y1 = y1 + lax.dot_general(
            _shift_rows(x, dk - 1), w1_ref[dk],
            (((1,), (0,)), ((), ())),
            preferred_element_type=jnp.float32, precision=prec)
    a = jnp.maximum(y1 + b1_ref[...], 0.0)

    y2 = jnp.zeros((S_NH, MEM_COMPRESS), jnp.float32)
    for dk in range(3):
        y2 = y2 + lax.dot_general(
            _shift_rows(a, dk - 1), w2_ref[dk],
            (((1,), (0,)), ((), ())),
            preferred_element_type=jnp.float32, precision=prec)
    y2 = y2 + b2_ref[...]

    mx = jnp.max(y2, axis=0, keepdims=True)
    e = jnp.exp(y2 - mx)
    soft = e / jnp.sum(e, axis=0, keepdims=True)

    tio = lax.broadcasted_iota(jnp.int32, (S_NH, MEM_COMPRESS), 0)
    oh = jnp.where(rw_ref[0] == tio, 1.0, 0.0)

    nrm = nrm_ref[0, 0]
    w = oh * (1.0 - nrm) + soft * nrm

    nk_ref[0] = lax.dot_general(
        w, x[:, :D], (((0,), (0,)), ((), ())),
        preferred_element_type=jnp.float32, precision=prec)
    nv_ref[0] = lax.dot_general(
        w, x[:, D:], (((0,), (0,)), ((), ())),
        preferred_element_type=jnp.float32, precision=prec)


def _conv_call(perm_kv, rw, w1t, w2t, b1, b2, nrm):
    return pl.pallas_call(
        _conv_body,
        grid=(NHEAD,),
        in_specs=[
            pl.BlockSpec((1, S, DIM_KV), lambda h: (h, 0, 0)),
            pl.BlockSpec((1, 1, MEM_COMPRESS), lambda h: (h, 0, 0)),
            pl.BlockSpec((3, DIM_KV, HIDDEN), lambda h: (0, 0, 0)),
            pl.BlockSpec((3, HIDDEN, MEM_COMPRESS), lambda h: (0, 0, 0)),
            pl.BlockSpec((1, HIDDEN), lambda h: (0, 0)),
            pl.BlockSpec((1, MEM_COMPRESS), lambda h: (0, 0)),
            pl.BlockSpec(memory_space=pltpu.SMEM),
        ],
        out_specs=[
            pl.BlockSpec((1, MEM_COMPRESS, D), lambda h: (h, 0, 0)),
            pl.BlockSpec((1, MEM_COMPRESS, D), lambda h: (h, 0, 0)),
        ],
        out_shape=[
            jax.ShapeDtypeStruct((NHEAD, MEM_COMPRESS, D), jnp.float32),
            jax.ShapeDtypeStruct((NHEAD, MEM_COMPRESS, D), jnp.float32),
        ],
        compiler_params=pltpu.CompilerParams(
            dimension_semantics=("arbitrary",)),
    )(perm_kv, rw, w1t, w2t, b1, b2, nrm)


def kernel(past_key_states, past_value_states, hh_scores, W1, b1, W2, b2,
           normalizer):
    scores = hh_scores.reshape(NHEAD, S)
    dest, rw = _index_call(scores)

    kv = jnp.concatenate(
        [past_key_states.reshape(ROWS, D),
         past_value_states.reshape(ROWS, D)], axis=1)
    perm_kv = _sc_scatter(kv, dest.reshape(NCHUNKS, CHUNK))

    w1t = jnp.transpose(W1, (2, 1, 0))
    w2t = jnp.transpose(W2, (2, 1, 0))
    nk, nv = _conv_call(
        perm_kv.reshape(NHEAD, S, DIM_KV), rw, w1t, w2t,
        b1.reshape(1, HIDDEN), b2.reshape(1, MEM_COMPRESS),
        normalizer.reshape(1, 1))

    perm4 = perm_kv.reshape(B, H, S, DIM_KV)
    k_out = jnp.concatenate(
        [perm4[:, :, :KEEP_HH, :D], nk.reshape(B, H, MEM_COMPRESS, D)], axis=2)
    v_out = jnp.concatenate(
        [perm4[:, :, :KEEP_HH, D:], nv.reshape(B, H, MEM_COMPRESS, D)], axis=2)
    return k_out, v_out


# speedup vs baseline: 2.7839x; 1.1672x over previous
import functools

import jax
import jax.numpy as jnp
from jax import lax
from jax.experimental import pallas as pl
from jax.experimental.pallas import tpu as pltpu
from jax.experimental.pallas import tpu_sc as plsc

B, H, S, D = 8, 12, 2048, 64
MEM_COMPRESS = 256
KEEP_HH = 256
LOCAL_LEN = 64
DIM_KV = 2 * D
HIDDEN = 512
S_NH = S - KEEP_HH
NHEAD = B * H
K1 = KEEP_HH - LOCAL_LEN
NLOC = S - LOCAL_LEN
ROWS = NHEAD * S
CHUNK = 128
NCHUNKS = ROWS // CHUNK

_HEADS_PER_BLK = 8


def _cumsum_lanes(x):
    n = x.shape[-1]
    s = 1
    while s < n:
        shifted = jnp.concatenate(
            [jnp.zeros(x.shape[:-1] + (s,), x.dtype), x[..., : n - s]], axis=-1)
        x = x + shifted
        s *= 2
    return x


def _topk_mask(bits, valid, k):
    rows = bits.shape[0]
    validc = valid.astype(jnp.int32)

    def count_ge(m):
        return jnp.sum(jnp.where(bits >= m, validc, 0), axis=-1, keepdims=True)

    def body(_, lohi):
        lo, hi = lohi
        mid = (lo + hi) // 2
        ge = count_ge(mid) >= k
        return (jnp.where(ge, mid, lo), jnp.where(ge, hi, mid))

    lo = jnp.zeros((rows, 1), jnp.int32)
    hi = jnp.full((rows, 1), 0x3F800000, jnp.int32)
    lo, hi = lax.fori_loop(0, 31, body, (lo, hi))
    t = lo
    gt = valid & (bits > t)
    eq = valid & (bits == t)
    need = k - jnp.sum(gt.astype(jnp.int32), axis=-1, keepdims=True)
    eqrank = _cumsum_lanes(eq.astype(jnp.int32))
    return gt | (eq & (eqrank <= need))


def _index_body(scores_ref, dest_ref, rw_ref):
    blk = lax.broadcasted_iota(jnp.int32, (_HEADS_PER_BLK, S), 0)
    head0 = pl.program_id(0) * _HEADS_PER_BLK
    bits = lax.bitcast_convert_type(scores_ref[...], jnp.int32)
    pos = lax.broadcasted_iota(jnp.int32, (_HEADS_PER_BLK, S), 1)

    mask1 = _topk_mask(bits, pos < NLOC, K1) | (pos >= NLOC)
    c1 = _cumsum_lanes(mask1.astype(jnp.int32))
    dest = jnp.where(mask1, c1 - 1, KEEP_HH + pos - c1)
    dest_ref[...] = dest + (head0 + blk) * S

    mask2 = _topk_mask(bits, ~mask1, MEM_COMPRESS)
    c2 = _cumsum_lanes(mask2.astype(jnp.int32))
    r = pos - c1
    mio = lax.broadcasted_iota(jnp.int32, (MEM_COMPRESS, S), 0)
    for h in range(_HEADS_PER_BLK):
        hit = mask2[h : h + 1, :] & (c2[h : h + 1, :] - 1 == mio)
        rw_ref[h, 0, :] = jnp.sum(
            jnp.where(hit, r[h : h + 1, :], 0), axis=1)


def _index_call(scores):
    grid = (NHEAD // _HEADS_PER_BLK,)
    return pl.pallas_call(
        _index_body,
        grid=grid,
        in_specs=[pl.BlockSpec((_HEADS_PER_BLK, S), lambda i: (i, 0))],
        out_specs=[
            pl.BlockSpec((_HEADS_PER_BLK, S), lambda i: (i, 0)),
            pl.BlockSpec((_HEADS_PER_BLK, 1, MEM_COMPRESS),
                         lambda i: (i, 0, 0)),
        ],
        out_shape=[
            jax.ShapeDtypeStruct((NHEAD, S), jnp.int32),
            jax.ShapeDtypeStruct((NHEAD, 1, MEM_COMPRESS), jnp.int32),
        ],
        compiler_params=pltpu.CompilerParams(
            dimension_semantics=("parallel",)),
    )(scores)


def _sc_scatter(kv, idx):
    info = plsc.get_sparse_core_info()
    nw = info.num_cores * info.num_subcores
    per_w = NCHUNKS // nw

    mesh = plsc.VectorSubcoreMesh(core_axis_name="c", subcore_axis_name="s")

    @functools.partial(
        pl.kernel,
        mesh=mesh,
        out_type=jax.ShapeDtypeStruct((ROWS, DIM_KV), jnp.float32),
        scratch_types=[
            pltpu.VMEM((CHUNK,), jnp.int32),
            pltpu.VMEM((CHUNK, DIM_KV), jnp.float32),
            pltpu.SemaphoreType.DMA,
        ],
    )
    def scatter(kv_hbm, idx_hbm, out_hbm, idx_v, buf, sem):
        wid = lax.axis_index("s") * info.num_cores + lax.axis_index("c")

        def body(j, carry):
            c = wid * per_w + j
            pltpu.sync_copy(idx_hbm.at[c], idx_v)
            pltpu.sync_copy(kv_hbm.at[pl.ds(c * CHUNK, CHUNK)], buf)
            pltpu.async_copy(buf, out_hbm.at[idx_v], sem).wait()
            return carry

        lax.fori_loop(0, per_w, body, 0)

    return scatter(kv, idx)


def _shift_rows(x, s):
    n = x.shape[0]
    if s == 0:
        return x
    if s > 0:
        return jnp.concatenate(
            [x[s:, :], jnp.zeros((s, x.shape[1]), x.dtype)], axis=0)
    return jnp.concatenate(
        [jnp.zeros((-s, x.shape[1]), x.dtype), x[: n + s, :]], axis=0)


def _conv_body(pkv_ref, rw_ref, w1_ref, w2_ref, b1_ref, b2_ref,
               nrm_ref, nk_ref, nv_ref):
    prec = lax.Precision.HIGHEST
    x = pkv_ref[0][KEEP_HH:, :]

    x3 = jnp.concatenate(
        [_shift_rows(x, -1), x, _shift_rows(x, 1)], axis=1)
    y1 = lax.dot_general(
        x3, w1_ref[...], (((1,), (0,)), ((), ())),
        preferred_element_type=jnp.float32, precision=prec)
    a = jnp.maximum(y1 + b1_ref[...], 0.0)

    y2 = jnp.zeros((S_NH, MEM_COMPRESS), jnp.float32)
    for dk in range(3):
        y2 = y2 + lax.dot_general(
            _shift_rows(a, dk - 1), w2_ref[dk],
            (((1,), (0,)), ((), ())),
            preferred_element_type=jnp.float32, precision=prec)
    y2 = y2 + b2_ref[...]

    mx = jnp.max(y2, axis=0, keepdims=True)
    e = jnp.exp(y2 - mx)
    soft = e / jnp.sum(e, axis=0, keepdims=True)

    tio = lax.broadcasted_iota(jnp.int32, (S_NH, MEM_COMPRESS), 0)
    oh = jnp.where(rw_ref[0] == tio, 1.0, 0.0)

    nrm = nrm_ref[0, 0]
    w = oh * (1.0 - nrm) + soft * nrm

    nkv = lax.dot_general(
        w, x, (((0,), (0,)), ((), ())),
        preferred_element_type=jnp.float32, precision=prec)
    nk_ref[0] = nkv[:, :D]
    nv_ref[0] = nkv[:, D:]


def _conv_call(perm_kv, rw, w1t, w2t, b1, b2, nrm):
    return pl.pallas_call(
        _conv_body,
        grid=(NHEAD,),
        in_specs=[
            pl.BlockSpec((1, S, DIM_KV), lambda h: (h, 0, 0)),
            pl.BlockSpec((1, 1, MEM_COMPRESS), lambda h: (h, 0, 0)),
            pl.BlockSpec((3 * DIM_KV, HIDDEN), lambda h: (0, 0)),
            pl.BlockSpec((3, HIDDEN, MEM_COMPRESS), lambda h: (0, 0, 0)),
            pl.BlockSpec((1, HIDDEN), lambda h: (0, 0)),
            pl.BlockSpec((1, MEM_COMPRESS), lambda h: (0, 0)),
            pl.BlockSpec(memory_space=pltpu.SMEM),
        ],
        out_specs=[
            pl.BlockSpec((1, MEM_COMPRESS, D), lambda h: (h, 0, 0)),
            pl.BlockSpec((1, MEM_COMPRESS, D), lambda h: (h, 0, 0)),
        ],
        out_shape=[
            jax.ShapeDtypeStruct((NHEAD, MEM_COMPRESS, D), jnp.float32),
            jax.ShapeDtypeStruct((NHEAD, MEM_COMPRESS, D), jnp.float32),
        ],
        compiler_params=pltpu.CompilerParams(
            dimension_semantics=("arbitrary",)),
    )(perm_kv, rw, w1t, w2t, b1, b2, nrm)


def kernel(past_key_states, past_value_states, hh_scores, W1, b1, W2, b2,
           normalizer):
    scores = hh_scores.reshape(NHEAD, S)
    dest, rw = _index_call(scores)

    kv = jnp.concatenate(
        [past_key_states.reshape(ROWS, D),
         past_value_states.reshape(ROWS, D)], axis=1)
    perm_kv = _sc_scatter(kv, dest.reshape(NCHUNKS, CHUNK))

    w1t = jnp.transpose(W1, (2, 1, 0)).reshape(3 * DIM_KV, HIDDEN)
    w2t = jnp.transpose(W2, (2, 1, 0))
    nk, nv = _conv_call(
        perm_kv.reshape(NHEAD, S, DIM_KV), rw, w1t, w2t,
        b1.reshape(1, HIDDEN), b2.reshape(1, MEM_COMPRESS),
        normalizer.reshape(1, 1))

    perm4 = perm_kv.reshape(B, H, S, DIM_KV)
    k_out = jnp.concatenate(
        [perm4[:, :, :KEEP_HH, :D], nk.reshape(B, H, MEM_COMPRESS, D)], axis=2)
    v_out = jnp.concatenate(
        [perm4[:, :, :KEEP_HH, D:], nv.reshape(B, H, MEM_COMPRESS, D)], axis=2)
    return k_out, v_out


# speedup vs baseline: 6.8974x; 2.4776x over previous
import functools

import jax
import jax.numpy as jnp
from jax import lax
from jax.experimental import pallas as pl
from jax.experimental.pallas import tpu as pltpu
from jax.experimental.pallas import tpu_sc as plsc

B, H, S, D = 8, 12, 2048, 64
MEM_COMPRESS = 256
KEEP_HH = 256
LOCAL_LEN = 64
DIM_KV = 2 * D
HIDDEN = 512
S_NH = S - KEEP_HH
NHEAD = B * H
K1 = KEEP_HH - LOCAL_LEN
NLOC = S - LOCAL_LEN
ROWS = NHEAD * S
CHUNK = 128
NCHUNKS = ROWS // CHUNK

_HEADS_PER_BLK = 8


def _cumsum_lanes(x):
    n = x.shape[-1]
    s = 1
    while s < n:
        shifted = jnp.concatenate(
            [jnp.zeros(x.shape[:-1] + (s,), x.dtype), x[..., : n - s]], axis=-1)
        x = x + shifted
        s *= 2
    return x


def _topk_mask(bits, valid, k):
    rows = bits.shape[0]
    validc = valid.astype(jnp.int32)

    def count_ge(m):
        return jnp.sum(jnp.where(bits >= m, validc, 0), axis=-1, keepdims=True)

    def body(_, lohi):
        lo, hi = lohi
        mid = (lo + hi) // 2
        ge = count_ge(mid) >= k
        return (jnp.where(ge, mid, lo), jnp.where(ge, hi, mid))

    lo = jnp.zeros((rows, 1), jnp.int32)
    hi = jnp.full((rows, 1), 0x3F800000, jnp.int32)
    lo, hi = lax.fori_loop(0, 31, body, (lo, hi))
    t = lo
    gt = valid & (bits > t)
    eq = valid & (bits == t)
    need = k - jnp.sum(gt.astype(jnp.int32), axis=-1, keepdims=True)
    eqrank = _cumsum_lanes(eq.astype(jnp.int32))
    return gt | (eq & (eqrank <= need))


def _index_body(scores_ref, dest_ref, rw_ref):
    blk = lax.broadcasted_iota(jnp.int32, (_HEADS_PER_BLK, S), 0)
    head0 = pl.program_id(0) * _HEADS_PER_BLK
    bits = lax.bitcast_convert_type(scores_ref[...], jnp.int32)
    pos = lax.broadcasted_iota(jnp.int32, (_HEADS_PER_BLK, S), 1)

    mask1 = _topk_mask(bits, pos < NLOC, K1) | (pos >= NLOC)
    c1 = _cumsum_lanes(mask1.astype(jnp.int32))
    dest = jnp.where(mask1, c1 - 1, KEEP_HH + pos - c1)
    dest_ref[...] = dest + (head0 + blk) * S

    mask2 = _topk_mask(bits, ~mask1, MEM_COMPRESS)
    c2 = _cumsum_lanes(mask2.astype(jnp.int32))
    r = pos - c1
    mio = lax.broadcasted_iota(jnp.int32, (MEM_COMPRESS, S), 0)
    for h in range(_HEADS_PER_BLK):
        hit = mask2[h : h + 1, :] & (c2[h : h + 1, :] - 1 == mio)
        rw_ref[h, 0, :] = jnp.sum(
            jnp.where(hit, r[h : h + 1, :], 0), axis=1)


def _index_call(scores):
    grid = (NHEAD // _HEADS_PER_BLK,)
    return pl.pallas_call(
        _index_body,
        grid=grid,
        in_specs=[pl.BlockSpec((_HEADS_PER_BLK, S), lambda i: (i, 0))],
        out_specs=[
            pl.BlockSpec((_HEADS_PER_BLK, S), lambda i: (i, 0)),
            pl.BlockSpec((_HEADS_PER_BLK, 1, MEM_COMPRESS),
                         lambda i: (i, 0, 0)),
        ],
        out_shape=[
            jax.ShapeDtypeStruct((NHEAD, S), jnp.int32),
            jax.ShapeDtypeStruct((NHEAD, 1, MEM_COMPRESS), jnp.int32),
        ],
        compiler_params=pltpu.CompilerParams(
            dimension_semantics=("parallel",)),
    )(scores)


def _sc_scatter(kv, idx):
    info = plsc.get_sparse_core_info()
    nw = info.num_cores * info.num_subcores
    per_w = NCHUNKS // nw

    mesh = plsc.VectorSubcoreMesh(core_axis_name="c", subcore_axis_name="s")

    @functools.partial(
        pl.kernel,
        mesh=mesh,
        out_type=jax.ShapeDtypeStruct((ROWS, DIM_KV), jnp.float32),
        scratch_types=[
            pltpu.VMEM((CHUNK,), jnp.int32),
            pltpu.VMEM((CHUNK,), jnp.int32),
            pltpu.VMEM((CHUNK, DIM_KV), jnp.float32),
            pltpu.VMEM((CHUNK, DIM_KV), jnp.float32),
            pltpu.SemaphoreType.DMA,
            pltpu.SemaphoreType.DMA,
            pltpu.SemaphoreType.DMA,
            pltpu.SemaphoreType.DMA,
        ],
    )
    def scatter(kv_hbm, idx_hbm, out_hbm, idx0, idx1, buf0, buf1,
                sl0, sl1, ss0, ss1):
        wid = lax.axis_index("s") * info.num_cores + lax.axis_index("c")
        base = wid * per_w
        bufs = ((idx0, buf0, sl0, ss0), (idx1, buf1, sl1, ss1))

        def loads(c, b):
            idxb, kvb, slb, _ = bufs[b]
            return (pltpu.make_async_copy(idx_hbm.at[c], idxb, slb),
                    pltpu.make_async_copy(
                        kv_hbm.at[pl.ds(c * CHUNK, CHUNK)], kvb, slb))

        def scat(b):
            idxb, kvb, _, ssb = bufs[b]
            return pltpu.make_async_copy(kvb, out_hbm.at[idxb], ssb)

        for d in loads(base, 0):
            d.start()

        def body(j, carry):
            c = base + j

            def step(b):
                for d in loads(c, b):
                    d.wait()

                @pl.when(j > 0)
                def _():
                    scat(1 - b).wait()

                @pl.when(j + 1 < per_w)
                def _():
                    for d in loads(c + 1, 1 - b):
                        d.start()

                scat(b).start()

            @pl.when(j % 2 == 0)
            def _():
                step(0)

            @pl.when(j % 2 == 1)
            def _():
                step(1)

            return carry

        lax.fori_loop(0, per_w, body, 0)
        scat((per_w - 1) % 2).wait()

    return scatter(kv, idx)


def _shift_rows(x, s):
    n = x.shape[0]
    if s == 0:
        return x
    if s > 0:
        return jnp.concatenate(
            [x[s:, :], jnp.zeros((s, x.shape[1]), x.dtype)], axis=0)
    return jnp.concatenate(
        [jnp.zeros((-s, x.shape[1]), x.dtype), x[: n + s, :]], axis=0)


def _conv_body(pkv_ref, rw_ref, w1_ref, w2_ref, b1_ref, b2_ref,
               nrm_ref, nk_ref, nv_ref):
    prec = lax.Precision.DEFAULT
    x = pkv_ref[0][KEEP_HH:, :]

    x3 = jnp.concatenate(
        [_shift_rows(x, -1), x, _shift_rows(x, 1)], axis=1)
    y1 = lax.dot_general(
        x3, w1_ref[...], (((1,), (0,)), ((), ())),
        preferred_element_type=jnp.float32, precision=prec)
    a = jnp.maximum(y1 + b1_ref[...], 0.0)

    y2 = jnp.zeros((S_NH, MEM_COMPRESS), jnp.float32)
    for dk in range(3):
        y2 = y2 + lax.dot_general(
            _shift_rows(a, dk - 1), w2_ref[dk],
            (((1,), (0,)), ((), ())),
            preferred_element_type=jnp.float32, precision=prec)
    y2 = y2 + b2_ref[...]

    mx = jnp.max(y2, axis=0, keepdims=True)
    e = jnp.exp(y2 - mx)
    soft = e / jnp.sum(e, axis=0, keepdims=True)

    tio = lax.broadcasted_iota(jnp.int32, (S_NH, MEM_COMPRESS), 0)
    oh = jnp.where(rw_ref[0] == tio, 1.0, 0.0)

    nrm = nrm_ref[0, 0]
    w = oh * (1.0 - nrm) + soft * nrm

    nkv = lax.dot_general(
        w, x, (((0,), (0,)), ((), ())),
        preferred_element_type=jnp.float32, precision=prec)
    nk_ref[0] = nkv[:, :D]
    nv_ref[0] = nkv[:, D:]


def _conv_call(perm_kv, rw, w1t, w2t, b1, b2, nrm):
    return pl.pallas_call(
        _conv_body,
        grid=(NHEAD,),
        in_specs=[
            pl.BlockSpec((1, S, DIM_KV), lambda h: (h, 0, 0)),
            pl.BlockSpec((1, 1, MEM_COMPRESS), lambda h: (h, 0, 0)),
            pl.BlockSpec((3 * DIM_KV, HIDDEN), lambda h: (0, 0)),
            pl.BlockSpec((3, HIDDEN, MEM_COMPRESS), lambda h: (0, 0, 0)),
            pl.BlockSpec((1, HIDDEN), lambda h: (0, 0)),
            pl.BlockSpec((1, MEM_COMPRESS), lambda h: (0, 0)),
            pl.BlockSpec(memory_space=pltpu.SMEM),
        ],
        out_specs=[
            pl.BlockSpec((1, MEM_COMPRESS, D), lambda h: (h, 0, 0)),
            pl.BlockSpec((1, MEM_COMPRESS, D), lambda h: (h, 0, 0)),
        ],
        out_shape=[
            jax.ShapeDtypeStruct((NHEAD, MEM_COMPRESS, D), jnp.float32),
            jax.ShapeDtypeStruct((NHEAD, MEM_COMPRESS, D), jnp.float32),
        ],
        compiler_params=pltpu.CompilerParams(
            dimension_semantics=("arbitrary",)),
    )(perm_kv, rw, w1t, w2t, b1, b2, nrm)


def kernel(past_key_states, past_value_states, hh_scores, W1, b1, W2, b2,
           normalizer):
    scores = hh_scores.reshape(NHEAD, S)
    dest, rw = _index_call(scores)

    kv = jnp.concatenate(
        [past_key_states.reshape(ROWS, D),
         past_value_states.reshape(ROWS, D)], axis=1)
    perm_kv = _sc_scatter(kv, dest.reshape(NCHUNKS, CHUNK))

    w1t = jnp.transpose(W1, (2, 1, 0)).reshape(3 * DIM_KV, HIDDEN)
    w2t = jnp.transpose(W2, (2, 1, 0))
    nk, nv = _conv_call(
        perm_kv.reshape(NHEAD, S, DIM_KV), rw, w1t, w2t,
        b1.reshape(1, HIDDEN), b2.reshape(1, MEM_COMPRESS),
        normalizer.reshape(1, 1))

    perm4 = perm_kv.reshape(B, H, S, DIM_KV)
    k_out = jnp.concatenate(
        [perm4[:, :, :KEEP_HH, :D], nk.reshape(B, H, MEM_COMPRESS, D)], axis=2)
    v_out = jnp.concatenate(
        [perm4[:, :, :KEEP_HH, D:], nv.reshape(B, H, MEM_COMPRESS, D)], axis=2)
    return k_out, v_out


# speedup vs baseline: 11.5552x; 1.6753x over previous
import functools

import jax
import jax.numpy as jnp
from jax import lax
from jax.experimental import pallas as pl
from jax.experimental.pallas import tpu as pltpu
from jax.experimental.pallas import tpu_sc as plsc

B, H, S, D = 8, 12, 2048, 64
MEM_COMPRESS = 256
KEEP_HH = 256
LOCAL_LEN = 64
DIM_KV = 2 * D
HIDDEN = 512
S_NH = S - KEEP_HH
NHEAD = B * H
K1 = KEEP_HH - LOCAL_LEN
NLOC = S - LOCAL_LEN
ROWS = NHEAD * S
CHUNK = 128
NCHUNKS = ROWS // CHUNK

_HEADS_PER_BLK = 8


def _cumsum_lanes(x):
    n = x.shape[-1]
    s = 1
    while s < n:
        shifted = jnp.concatenate(
            [jnp.zeros(x.shape[:-1] + (s,), x.dtype), x[..., : n - s]], axis=-1)
        x = x + shifted
        s *= 2
    return x


def _topk_mask(bits, valid, k):
    rows = bits.shape[0]
    bitsm = jnp.where(valid, bits, -1)

    def count_ge(m):
        return jnp.sum((bitsm >= m).astype(jnp.int32), axis=-1, keepdims=True)

    def body(_, lohi):
        lo, hi = lohi
        mid = (lo + hi) // 2
        ge = count_ge(mid) >= k
        return (jnp.where(ge, mid, lo), jnp.where(ge, hi, mid))

    lo = jnp.zeros((rows, 1), jnp.int32)
    hi = jnp.full((rows, 1), 0x3F800000, jnp.int32)
    lo, hi = lax.fori_loop(0, 31, body, (lo, hi))
    t = lo
    gt = bitsm > t
    eq = bitsm == t
    need = k - jnp.sum(gt.astype(jnp.int32), axis=-1, keepdims=True)
    eqrank = _cumsum_lanes(eq.astype(jnp.int32))
    return gt | (eq & (eqrank <= need))


def _index_body(scores_ref, dest_ref, rw_ref):
    blk = lax.broadcasted_iota(jnp.int32, (_HEADS_PER_BLK, S), 0)
    head0 = pl.program_id(0) * _HEADS_PER_BLK
    bits = lax.bitcast_convert_type(scores_ref[...], jnp.int32)
    pos = lax.broadcasted_iota(jnp.int32, (_HEADS_PER_BLK, S), 1)

    mask1 = _topk_mask(bits, pos < NLOC, K1) | (pos >= NLOC)
    c1 = _cumsum_lanes(mask1.astype(jnp.int32))
    dest = jnp.where(mask1, c1 - 1, KEEP_HH + pos - c1)
    dest_ref[...] = dest + (head0 + blk) * S

    mask2 = _topk_mask(bits, ~mask1, MEM_COMPRESS)
    c2 = _cumsum_lanes(mask2.astype(jnp.int32))
    r = pos - c1
    c2m = jnp.where(mask2, c2 - 1, -1)
    rf = r.astype(jnp.float32)
    mio = lax.broadcasted_iota(jnp.int32, (MEM_COMPRESS, S), 0)
    for h in range(_HEADS_PER_BLK):
        hit = (c2m[h : h + 1, :] == mio).astype(jnp.float32)
        rw_ref[h, 0, :] = lax.dot_general(
            hit, rf[h : h + 1, :], (((1,), (1,)), ((), ())),
            preferred_element_type=jnp.float32,
            precision=lax.Precision.HIGHEST)[:, 0].astype(jnp.int32)


def _index_call(scores):
    grid = (NHEAD // _HEADS_PER_BLK,)
    return pl.pallas_call(
        _index_body,
        grid=grid,
        in_specs=[pl.BlockSpec((_HEADS_PER_BLK, S), lambda i: (i, 0))],
        out_specs=[
            pl.BlockSpec((_HEADS_PER_BLK, S), lambda i: (i, 0)),
            pl.BlockSpec((_HEADS_PER_BLK, 1, MEM_COMPRESS),
                         lambda i: (i, 0, 0)),
        ],
        out_shape=[
            jax.ShapeDtypeStruct((NHEAD, S), jnp.int32),
            jax.ShapeDtypeStruct((NHEAD, 1, MEM_COMPRESS), jnp.int32),
        ],
        compiler_params=pltpu.CompilerParams(
            dimension_semantics=("parallel",)),
    )(scores)


def _sc_scatter(kv, idx):
    info = plsc.get_sparse_core_info()
    nw = info.num_cores * info.num_subcores
    per_w = NCHUNKS // nw

    mesh = plsc.VectorSubcoreMesh(core_axis_name="c", subcore_axis_name="s")

    @functools.partial(
        pl.kernel,
        mesh=mesh,
        out_type=jax.ShapeDtypeStruct((ROWS, DIM_KV), jnp.float32),
        scratch_types=[
            pltpu.VMEM((CHUNK,), jnp.int32),
            pltpu.VMEM((CHUNK,), jnp.int32),
            pltpu.VMEM((CHUNK, DIM_KV), jnp.float32),
            pltpu.VMEM((CHUNK, DIM_KV), jnp.float32),
            pltpu.SemaphoreType.DMA,
            pltpu.SemaphoreType.DMA,
            pltpu.SemaphoreType.DMA,
            pltpu.SemaphoreType.DMA,
        ],
    )
    def scatter(kv_hbm, idx_hbm, out_hbm, idx0, idx1, buf0, buf1,
                sl0, sl1, ss0, ss1):
        wid = lax.axis_index("s") * info.num_cores + lax.axis_index("c")
        base = wid * per_w
        bufs = ((idx0, buf0, sl0, ss0), (idx1, buf1, sl1, ss1))

        def loads(c, b):
            idxb, kvb, slb, _ = bufs[b]
            return (pltpu.make_async_copy(idx_hbm.at[c], idxb, slb),
                    pltpu.make_async_copy(
                        kv_hbm.at[pl.ds(c * CHUNK, CHUNK)], kvb, slb))

        def scat(b):
            idxb, kvb, _, ssb = bufs[b]
            return pltpu.make_async_copy(kvb, out_hbm.at[idxb], ssb)

        for d in loads(base, 0):
            d.start()

        def body(j, carry):
            c = base + j

            def step(b):
                for d in loads(c, b):
                    d.wait()

                @pl.when(j > 0)
                def _():
                    scat(1 - b).wait()

                @pl.when(j + 1 < per_w)
                def _():
                    for d in loads(c + 1, 1 - b):
                        d.start()

                scat(b).start()

            @pl.when(j % 2 == 0)
            def _():
                step(0)

            @pl.when(j % 2 == 1)
            def _():
                step(1)

            return carry

        lax.fori_loop(0, per_w, body, 0)
        scat((per_w - 1) % 2).wait()

    return scatter(kv, idx)


def _shift_rows(x, s):
    n = x.shape[0]
    if s == 0:
        return x
    if s > 0:
        return jnp.concatenate(
            [x[s:, :], jnp.zeros((s, x.shape[1]), x.dtype)], axis=0)
    return jnp.concatenate(
        [jnp.zeros((-s, x.shape[1]), x.dtype), x[: n + s, :]], axis=0)


def _conv_body(pkv_ref, rw_ref, w1_ref, w2_ref, b1_ref, b2_ref,
               nrm_ref, nk_ref, nv_ref):
    prec = lax.Precision.DEFAULT
    x = pkv_ref[0][KEEP_HH:, :]

    x3 = jnp.concatenate(
        [_shift_rows(x, -1), x, _shift_rows(x, 1)], axis=1)
    y1 = lax.dot_general(
        x3, w1_ref[...], (((1,), (0,)), ((), ())),
        preferred_element_type=jnp.float32, precision=prec)
    a = jnp.maximum(y1 + b1_ref[...], 0.0)

    y2 = jnp.zeros((S_NH, MEM_COMPRESS), jnp.float32)
    for dk in range(3):
        y2 = y2 + lax.dot_general(
            _shift_rows(a, dk - 1), w2_ref[dk],
            (((1,), (0,)), ((), ())),
            preferred_element_type=jnp.float32, precision=prec)
    y2 = y2 + b2_ref[...]

    mx = jnp.max(y2, axis=0, keepdims=True)
    e = jnp.exp(y2 - mx)
    soft = e / jnp.sum(e, axis=0, keepdims=True)

    tio = lax.broadcasted_iota(jnp.int32, (S_NH, MEM_COMPRESS), 0)
    oh = jnp.where(rw_ref[0] == tio, 1.0, 0.0)

    nrm = nrm_ref[0, 0]
    w = oh * (1.0 - nrm) + soft * nrm

    nkv = lax.dot_general(
        w, x, (((0,), (0,)), ((), ())),
        preferred_element_type=jnp.float32, precision=prec)
    nk_ref[0] = nkv[:, :D]
    nv_ref[0] = nkv[:, D:]


def _conv_call(perm_kv, rw, w1t, w2t, b1, b2, nrm):
    return pl.pallas_call(
        _conv_body,
        grid=(NHEAD,),
        in_specs=[
            pl.BlockSpec((1, S, DIM_KV), lambda h: (h, 0, 0)),
            pl.BlockSpec((1, 1, MEM_COMPRESS), lambda h: (h, 0, 0)),
            pl.BlockSpec((3 * DIM_KV, HIDDEN), lambda h: (0, 0)),
            pl.BlockSpec((3, HIDDEN, MEM_COMPRESS), lambda h: (0, 0, 0)),
            pl.BlockSpec((1, HIDDEN), lambda h: (0, 0)),
            pl.BlockSpec((1, MEM_COMPRESS), lambda h: (0, 0)),
            pl.BlockSpec(memory_space=pltpu.SMEM),
        ],
        out_specs=[
            pl.BlockSpec((1, MEM_COMPRESS, D), lambda h: (h, 0, 0)),
            pl.BlockSpec((1, MEM_COMPRESS, D), lambda h: (h, 0, 0)),
        ],
        out_shape=[
            jax.ShapeDtypeStruct((NHEAD, MEM_COMPRESS, D), jnp.float32),
            jax.ShapeDtypeStruct((NHEAD, MEM_COMPRESS, D), jnp.float32),
        ],
        compiler_params=pltpu.CompilerParams(
            dimension_semantics=("arbitrary",)),
    )(perm_kv, rw, w1t, w2t, b1, b2, nrm)


def kernel(past_key_states, past_value_states, hh_scores, W1, b1, W2, b2,
           normalizer):
    scores = hh_scores.reshape(NHEAD, S)
    dest, rw = _index_call(scores)

    kv = jnp.concatenate(
        [past_key_states.reshape(ROWS, D),
         past_value_states.reshape(ROWS, D)], axis=1)
    perm_kv = _sc_scatter(kv, dest.reshape(NCHUNKS, CHUNK))

    w1t = jnp.transpose(W1, (2, 1, 0)).reshape(3 * DIM_KV, HIDDEN)
    w2t = jnp.transpose(W2, (2, 1, 0))
    nk, nv = _conv_call(
        perm_kv.reshape(NHEAD, S, DIM_KV), rw, w1t, w2t,
        b1.reshape(1, HIDDEN), b2.reshape(1, MEM_COMPRESS),
        normalizer.reshape(1, 1))

    perm4 = perm_kv.reshape(B, H, S, DIM_KV)
    k_out = jnp.concatenate(
        [perm4[:, :, :KEEP_HH, :D], nk.reshape(B, H, MEM_COMPRESS, D)], axis=2)
    v_out = jnp.concatenate(
        [perm4[:, :, :KEEP_HH, D:], nv.reshape(B, H, MEM_COMPRESS, D)], axis=2)
    return k_out, v_out


# speedup vs baseline: 13.2435x; 1.1461x over previous
import functools

import jax
import jax.numpy as jnp
from jax import lax
from jax.experimental import pallas as pl
from jax.experimental.pallas import tpu as pltpu
from jax.experimental.pallas import tpu_sc as plsc

B, H, S, D = 8, 12, 2048, 64
MEM_COMPRESS = 256
KEEP_HH = 256
LOCAL_LEN = 64
DIM_KV = 2 * D
HIDDEN = 512
S_NH = S - KEEP_HH
NHEAD = B * H
K1 = KEEP_HH - LOCAL_LEN
NLOC = S - LOCAL_LEN
ROWS = NHEAD * S
CHUNK = 128
NCHUNKS = ROWS // CHUNK

_HEADS_PER_BLK = 8


def _cumsum_lanes(x):
    n = x.shape[-1]
    s = 1
    while s < n:
        shifted = jnp.concatenate(
            [jnp.zeros(x.shape[:-1] + (s,), x.dtype), x[..., : n - s]], axis=-1)
        x = x + shifted
        s *= 2
    return x


def _topk_mask(bits, valid, k):
    rows = bits.shape[0]
    bitsm = jnp.where(valid, bits, -1)

    def count_ge(m):
        return jnp.sum((bitsm >= m).astype(jnp.int32), axis=-1, keepdims=True)

    def body(_, lohi):
        lo, hi = lohi
        mid = (lo + hi) // 2
        ge = count_ge(mid) >= k
        return (jnp.where(ge, mid, lo), jnp.where(ge, hi, mid))

    lo = jnp.zeros((rows, 1), jnp.int32)
    hi = jnp.full((rows, 1), 0x3F800000, jnp.int32)
    lo, hi = lax.fori_loop(0, 31, body, (lo, hi))
    t = lo
    gt = bitsm > t
    eq = bitsm == t
    need = k - jnp.sum(gt.astype(jnp.int32), axis=-1, keepdims=True)
    eqrank = _cumsum_lanes(eq.astype(jnp.int32))
    return gt | (eq & (eqrank <= need))


def _index_body(scores_ref, dest_ref, rw_ref):
    blk = lax.broadcasted_iota(jnp.int32, (_HEADS_PER_BLK, S), 0)
    head0 = pl.program_id(0) * _HEADS_PER_BLK
    bits = lax.bitcast_convert_type(scores_ref[...], jnp.int32)
    pos = lax.broadcasted_iota(jnp.int32, (_HEADS_PER_BLK, S), 1)

    mask1 = _topk_mask(bits, pos < NLOC, K1) | (pos >= NLOC)
    c1 = _cumsum_lanes(mask1.astype(jnp.int32))
    dest = jnp.where(mask1, c1 - 1, KEEP_HH + pos - c1)
    dest_ref[...] = dest + (head0 + blk) * S

    mask2 = _topk_mask(bits, ~mask1, MEM_COMPRESS)
    c2 = _cumsum_lanes(mask2.astype(jnp.int32))
    r = pos - c1
    c2m = jnp.where(mask2, c2 - 1, -1)
    rf = r.astype(jnp.float32)
    mio = lax.broadcasted_iota(jnp.int32, (MEM_COMPRESS, S), 0)
    for h in range(_HEADS_PER_BLK):
        hit = (c2m[h : h + 1, :] == mio).astype(jnp.float32)
        rw_ref[h, 0, :] = lax.dot_general(
            hit, rf[h : h + 1, :], (((1,), (1,)), ((), ())),
            preferred_element_type=jnp.float32,
            precision=lax.Precision.HIGHEST)[:, 0].astype(jnp.int32)


def _index_call(scores):
    grid = (NHEAD // _HEADS_PER_BLK,)
    return pl.pallas_call(
        _index_body,
        grid=grid,
        in_specs=[pl.BlockSpec((_HEADS_PER_BLK, S), lambda i: (i, 0))],
        out_specs=[
            pl.BlockSpec((_HEADS_PER_BLK, S), lambda i: (i, 0)),
            pl.BlockSpec((_HEADS_PER_BLK, 1, MEM_COMPRESS),
                         lambda i: (i, 0, 0)),
        ],
        out_shape=[
            jax.ShapeDtypeStruct((NHEAD, S), jnp.int32),
            jax.ShapeDtypeStruct((NHEAD, 1, MEM_COMPRESS), jnp.int32),
        ],
        compiler_params=pltpu.CompilerParams(
            dimension_semantics=("parallel",)),
    )(scores)


def _sc_scatter(kv, idx):
    info = plsc.get_sparse_core_info()
    nw = info.num_cores * info.num_subcores
    per_w = NCHUNKS // nw

    mesh = plsc.VectorSubcoreMesh(core_axis_name="c", subcore_axis_name="s")

    @functools.partial(
        pl.kernel,
        mesh=mesh,
        out_type=jax.ShapeDtypeStruct((ROWS, DIM_KV), jnp.float32),
        scratch_types=[
            pltpu.VMEM((CHUNK,), jnp.int32),
            pltpu.VMEM((CHUNK,), jnp.int32),
            pltpu.VMEM((CHUNK, DIM_KV), jnp.float32),
            pltpu.VMEM((CHUNK, DIM_KV), jnp.float32),
            pltpu.SemaphoreType.DMA,
            pltpu.SemaphoreType.DMA,
            pltpu.SemaphoreType.DMA,
            pltpu.SemaphoreType.DMA,
        ],
    )
    def scatter(kv_hbm, idx_hbm, out_hbm, idx0, idx1, buf0, buf1,
                sl0, sl1, ss0, ss1):
        wid = lax.axis_index("s") * info.num_cores + lax.axis_index("c")
        base = wid * per_w
        bufs = ((idx0, buf0, sl0, ss0), (idx1, buf1, sl1, ss1))

        def loads(c, b):
            idxb, kvb, slb, _ = bufs[b]
            return (pltpu.make_async_copy(idx_hbm.at[c], idxb, slb),
                    pltpu.make_async_copy(
                        kv_hbm.at[pl.ds(c * CHUNK, CHUNK)], kvb, slb))

        def scat(b):
            idxb, kvb, _, ssb = bufs[b]
            return pltpu.make_async_copy(kvb, out_hbm.at[idxb], ssb)

        for d in loads(base, 0):
            d.start()

        def body(j, carry):
            c = base + j

            def step(b):
                for d in loads(c, b):
                    d.wait()

                @pl.when(j > 0)
                def _():
                    scat(1 - b).wait()

                @pl.when(j + 1 < per_w)
                def _():
                    for d in loads(c + 1, 1 - b):
                        d.start()

                scat(b).start()

            @pl.when(j % 2 == 0)
            def _():
                step(0)

            @pl.when(j % 2 == 1)
            def _():
                step(1)

            return carry

        lax.fori_loop(0, per_w, body, 0)
        scat((per_w - 1) % 2).wait()

    return scatter(kv, idx)


def _shift_rows(x, s):
    n = x.shape[0]
    if s == 0:
        return x
    if s > 0:
        return jnp.concatenate(
            [x[s:, :], jnp.zeros((s, x.shape[1]), x.dtype)], axis=0)
    return jnp.concatenate(
        [jnp.zeros((-s, x.shape[1]), x.dtype), x[: n + s, :]], axis=0)


def _conv_body(pkv_ref, rw_ref, w1_ref, w2_ref, b1_ref, b2_ref,
               nrm_ref, nk_ref, nv_ref):
    prec = lax.Precision.DEFAULT
    x = pkv_ref[0][KEEP_HH:, :]

    x3 = jnp.concatenate(
        [_shift_rows(x, -1), x, _shift_rows(x, 1)], axis=1)
    y1 = lax.dot_general(
        x3, w1_ref[...], (((1,), (0,)), ((), ())),
        preferred_element_type=jnp.float32, precision=prec)
    a = jnp.maximum(y1 + b1_ref[...], 0.0)

    y2 = jnp.zeros((S_NH, MEM_COMPRESS), jnp.float32)
    for dk in range(3):
        p = lax.dot_general(
            a, w2_ref[dk], (((1,), (0,)), ((), ())),
            preferred_element_type=jnp.float32, precision=prec)
        y2 = y2 + _shift_rows(p, dk - 1)
    y2 = y2 + b2_ref[...]

    mx = jnp.max(y2, axis=0, keepdims=True)
    e = jnp.exp(y2 - mx)
    soft = e / jnp.sum(e, axis=0, keepdims=True)

    tio = lax.broadcasted_iota(jnp.int32, (S_NH, MEM_COMPRESS), 0)
    nrm = nrm_ref[0, 0]
    ws = soft * nrm
    w = jnp.where(rw_ref[0] == tio, ws + (1.0 - nrm), ws)

    nkv = lax.dot_general(
        w, x, (((0,), (0,)), ((), ())),
        preferred_element_type=jnp.float32, precision=prec)
    nk_ref[0] = nkv[:, :D]
    nv_ref[0] = nkv[:, D:]


def _conv_call(perm_kv, rw, w1t, w2t, b1, b2, nrm):
    return pl.pallas_call(
        _conv_body,
        grid=(NHEAD,),
        in_specs=[
            pl.BlockSpec((1, S, DIM_KV), lambda h: (h, 0, 0)),
            pl.BlockSpec((1, 1, MEM_COMPRESS), lambda h: (h, 0, 0)),
            pl.BlockSpec((3 * DIM_KV, HIDDEN), lambda h: (0, 0)),
            pl.BlockSpec((3, HIDDEN, MEM_COMPRESS), lambda h: (0, 0, 0)),
            pl.BlockSpec((1, HIDDEN), lambda h: (0, 0)),
            pl.BlockSpec((1, MEM_COMPRESS), lambda h: (0, 0)),
            pl.BlockSpec(memory_space=pltpu.SMEM),
        ],
        out_specs=[
            pl.BlockSpec((1, MEM_COMPRESS, D), lambda h: (h, 0, 0)),
            pl.BlockSpec((1, MEM_COMPRESS, D), lambda h: (h, 0, 0)),
        ],
        out_shape=[
            jax.ShapeDtypeStruct((NHEAD, MEM_COMPRESS, D), jnp.float32),
            jax.ShapeDtypeStruct((NHEAD, MEM_COMPRESS, D), jnp.float32),
        ],
        compiler_params=pltpu.CompilerParams(
            dimension_semantics=("arbitrary",)),
    )(perm_kv, rw, w1t, w2t, b1, b2, nrm)


def kernel(past_key_states, past_value_states, hh_scores, W1, b1, W2, b2,
           normalizer):
    scores = hh_scores.reshape(NHEAD, S)
    dest, rw = _index_call(scores)

    kv = jnp.concatenate(
        [past_key_states.reshape(ROWS, D),
         past_value_states.reshape(ROWS, D)], axis=1)
    perm_kv = _sc_scatter(kv, dest.reshape(NCHUNKS, CHUNK))

    w1t = jnp.transpose(W1, (2, 1, 0)).reshape(3 * DIM_KV, HIDDEN)
    w2t = jnp.transpose(W2, (2, 1, 0))
    nk, nv = _conv_call(
        perm_kv.reshape(NHEAD, S, DIM_KV), rw, w1t, w2t,
        b1.reshape(1, HIDDEN), b2.reshape(1, MEM_COMPRESS),
        normalizer.reshape(1, 1))

    perm4 = perm_kv.reshape(B, H, S, DIM_KV)
    k_out = jnp.concatenate(
        [perm4[:, :, :KEEP_HH, :D], nk.reshape(B, H, MEM_COMPRESS, D)], axis=2)
    v_out = jnp.concatenate(
        [perm4[:, :, :KEEP_HH, D:], nv.reshape(B, H, MEM_COMPRESS, D)], axis=2)
    return k_out, v_out
